# Initial kernel scaffold; baseline (speedup 1.0000x reference)
import jax
import jax.numpy as jnp
from jax.experimental import pallas as pl


def kernel(x, W_route, b_route, W_noise, b_noise, eps):
    raise NotImplementedError("write your pallas kernel here")


# speedup vs baseline: 2.3111x; 2.3111x over previous
import jax
import jax.numpy as jnp
from jax import lax
from jax.experimental import pallas as pl

N_TOKENS = 8192
D_MODEL = 2048
NUM_EXPERTS = 16
TOP_K = 2

BLOCK_T = 512


def _router_body(x_ref, wr_ref, br_ref, wn_ref, bn_ref, eps_ref,
                 out_ref, idx_ref):
    xb = x_ref[...]
    logits = jnp.dot(xb, wr_ref[...], preferred_element_type=jnp.float32)
    logits = logits + br_ref[...]
    nlogits = jnp.dot(xb, wn_ref[...], preferred_element_type=jnp.float32)
    nlogits = nlogits + bn_ref[...]
    noisy = logits + eps_ref[...] * jax.nn.softplus(nlogits)

    iota = lax.broadcasted_iota(jnp.int32, noisy.shape, 1)
    m1 = jnp.max(noisy, axis=1, keepdims=True)
    i1 = jnp.min(jnp.where(noisy == m1, iota, NUM_EXPERTS), axis=1,
                 keepdims=True)
    masked = jnp.where(iota == i1, -jnp.inf, noisy)
    m2 = jnp.max(masked, axis=1, keepdims=True)
    i2 = jnp.min(jnp.where(masked == m2, iota, NUM_EXPERTS), axis=1,
                 keepdims=True)
    keep = (iota == i1) | (iota == i2)
    z = jnp.where(keep, jnp.exp(noisy - m1), 0.0)
    out_ref[...] = z / jnp.sum(z, axis=1, keepdims=True)
    idx_ref[...] = jnp.concatenate([i1, i2], axis=1)


def kernel(x, W_route, b_route, W_noise, b_noise, eps):
    n_blocks = N_TOKENS // BLOCK_T
    br = b_route.reshape(1, NUM_EXPERTS)
    bn = b_noise.reshape(1, NUM_EXPERTS)
    out_shapes = (
        jax.ShapeDtypeStruct((N_TOKENS, NUM_EXPERTS), jnp.float32),
        jax.ShapeDtypeStruct((N_TOKENS, TOP_K), jnp.int32),
    )
    router_output, topk_indices = pl.pallas_call(
        _router_body,
        grid=(n_blocks,),
        in_specs=[
            pl.BlockSpec((BLOCK_T, D_MODEL), lambda i: (i, 0)),
            pl.BlockSpec((D_MODEL, NUM_EXPERTS), lambda i: (0, 0)),
            pl.BlockSpec((1, NUM_EXPERTS), lambda i: (0, 0)),
            pl.BlockSpec((D_MODEL, NUM_EXPERTS), lambda i: (0, 0)),
            pl.BlockSpec((1, NUM_EXPERTS), lambda i: (0, 0)),
            pl.BlockSpec((BLOCK_T, NUM_EXPERTS), lambda i: (i, 0)),
        ],
        out_specs=(
            pl.BlockSpec((BLOCK_T, NUM_EXPERTS), lambda i: (i, 0)),
            pl.BlockSpec((BLOCK_T, TOP_K), lambda i: (i, 0)),
        ),
        out_shape=out_shapes,
    )(x, W_route, br, W_noise, bn, eps)
    return (router_output, topk_indices)


# speedup vs baseline: 2.5969x; 1.1237x over previous
import jax
import jax.numpy as jnp
from jax import lax
from jax.experimental import pallas as pl

N_TOKENS = 8192
D_MODEL = 2048
NUM_EXPERTS = 16
TOP_K = 2

BLOCK_T = 1024


def _router_body(x_ref, w_ref, b_ref, eps_ref, out_ref, idx_ref):
    xb = x_ref[...]
    both = jnp.dot(xb, w_ref[...], preferred_element_type=jnp.float32)
    both = both + b_ref[...]
    logits = both[:, :NUM_EXPERTS]
    nlogits = both[:, NUM_EXPERTS:]
    noisy = logits + eps_ref[...] * jax.nn.softplus(nlogits)

    iota = lax.broadcasted_iota(jnp.int32, noisy.shape, 1)
    m1 = jnp.max(noisy, axis=1, keepdims=True)
    i1 = jnp.min(jnp.where(noisy == m1, iota, NUM_EXPERTS), axis=1,
                 keepdims=True)
    masked = jnp.where(iota == i1, -jnp.inf, noisy)
    m2 = jnp.max(masked, axis=1, keepdims=True)
    i2 = jnp.min(jnp.where(masked == m2, iota, NUM_EXPERTS), axis=1,
                 keepdims=True)
    keep = (iota == i1) | (iota == i2)
    z = jnp.where(keep, jnp.exp(noisy - m1), 0.0)
    out_ref[...] = z / jnp.sum(z, axis=1, keepdims=True)
    idx_ref[...] = jnp.concatenate([i1, i2], axis=1)


def kernel(x, W_route, b_route, W_noise, b_noise, eps):
    n_blocks = N_TOKENS // BLOCK_T
    w_cat = jnp.concatenate([W_route, W_noise], axis=1)
    b_cat = jnp.concatenate([b_route, b_noise]).reshape(1, 2 * NUM_EXPERTS)
    out_shapes = (
        jax.ShapeDtypeStruct((N_TOKENS, NUM_EXPERTS), jnp.float32),
        jax.ShapeDtypeStruct((N_TOKENS, TOP_K), jnp.int32),
    )
    router_output, topk_indices = pl.pallas_call(
        _router_body,
        grid=(n_blocks,),
        in_specs=[
            pl.BlockSpec((BLOCK_T, D_MODEL), lambda i: (i, 0)),
            pl.BlockSpec((D_MODEL, 2 * NUM_EXPERTS), lambda i: (0, 0)),
            pl.BlockSpec((1, 2 * NUM_EXPERTS), lambda i: (0, 0)),
            pl.BlockSpec((BLOCK_T, NUM_EXPERTS), lambda i: (i, 0)),
        ],
        out_specs=(
            pl.BlockSpec((BLOCK_T, NUM_EXPERTS), lambda i: (i, 0)),
            pl.BlockSpec((BLOCK_T, TOP_K), lambda i: (i, 0)),
        ),
        out_shape=out_shapes,
    )(x, w_cat, b_cat, eps)
    return (router_output, topk_indices)
